# baseline (speedup 1.0000x reference)
import functools

import jax
import jax.numpy as jnp
from jax import lax
from jax.experimental import pallas as pl
from jax.experimental.pallas import tpu as pltpu
from jax.experimental.pallas import tpu_sc as plsc

B, N, E = 2, 2048, 32768
D, H, DE = 256, 4, 16
OUT = 256
BH = B * H
NN = N * N
WST = (N + 1) * N
KST = (N + 1) * N
NTILE = 16
EPT = E // NTILE
NCH = EPT // 128
ZB = 16384
SEG = H * WST // NTILE
NZD = SEG // ZB
ZREM = SEG - NZD * ZB

_INV_SQRT_N = float(N) ** -0.5


BN = 512


def _proj_body(nf_ref, wq_ref, bq_ref, wk_ref, bk_ref, wv_ref, bv_ref,
               q_ref, k_ref, v_ref, vt_ref):
    x = nf_ref[0]
    q = jnp.dot(x, wq_ref[0], preferred_element_type=jnp.float32) + bq_ref[0]
    k = jnp.dot(x, wk_ref[0], preferred_element_type=jnp.float32) + bk_ref[0]
    v = jnp.dot(x, wv_ref[0], preferred_element_type=jnp.float32) + bv_ref[0]
    q_ref[0] = q
    k_ref[0] = k
    v_ref[0] = v
    m = pl.program_id(1)
    colsum = jnp.sum(v, axis=0, keepdims=True)

    @pl.when(m == 0)
    def _():
        vt_ref[0] = colsum

    @pl.when(m > 0)
    def _():
        vt_ref[0] = vt_ref[0] + colsum


def _tc1(nf, wq, bq3, wk, bk3, wv, bv3):
    grid = (BH, N // BN)
    hspec = lambda: pl.BlockSpec((1, D, D), lambda bh, m: (bh % H, 0, 0))
    bspec = lambda: pl.BlockSpec((1, 1, D), lambda bh, m: (bh % H, 0, 0))
    return pl.pallas_call(
        _proj_body,
        grid=grid,
        in_specs=[
            pl.BlockSpec((1, BN, D), lambda bh, m: (bh // H, m, 0)),
            hspec(), bspec(), hspec(), bspec(), hspec(), bspec(),
        ],
        out_specs=[
            pl.BlockSpec((1, BN, D), lambda bh, m: (bh, m, 0)),
            pl.BlockSpec((1, BN, D), lambda bh, m: (bh, m, 0)),
            pl.BlockSpec((1, BN, D), lambda bh, m: (bh, m, 0)),
            pl.BlockSpec((1, 1, D), lambda bh, m: (bh, 0, 0)),
        ],
        out_shape=[
            jax.ShapeDtypeStruct((BH, N, D), jnp.float32),
            jax.ShapeDtypeStruct((BH, N, D), jnp.float32),
            jax.ShapeDtypeStruct((BH, N, D), jnp.float32),
            jax.ShapeDtypeStruct((BH, 1, D), jnp.float32),
        ],
        compiler_params=pltpu.CompilerParams(
            dimension_semantics=("arbitrary", "arbitrary")),
    )(nf, wq, bq3, wk, bk3, wv, bv3)


def _csum_body(na_ref, c_ref):
    c_ref[0, 0] = jnp.sum(na_ref[0], axis=1) * _INV_SQRT_N


def _tcc(na):
    return pl.pallas_call(
        _csum_body,
        grid=(B,),
        in_specs=[pl.BlockSpec((1, E, DE), lambda b: (b, 0, 0))],
        out_specs=pl.BlockSpec((1, 1, E), lambda b: (b, 0, 0)),
        out_shape=jax.ShapeDtypeStruct((B, 1, E), jnp.float32),
    )(na)


def _edge_body(src_hbm, dst_hbm, c_hbm, q_hbm, k_hbm,
               w_hbm, win_hbm,
               srcb, dstb, cb, keyb, myidb, widb, scat, gq, gk, wkb,
               qr, kr, wb, zb,
               semz, sem1, semq, semk):
    b = lax.axis_index("c")
    t = lax.axis_index("s")
    lane = lax.iota(jnp.int32, 16)
    zero16 = jnp.zeros((16,), jnp.float32)
    _gd = lax.GatherDimensionNumbers(
        offset_dims=(), collapsed_slice_dims=(0,), start_index_map=(0,))
    perms = [(lane ^ m)[:, None] for m in (8, 4, 2, 1)]
    eqm = [lane == e for e in range(16)]

    def _lanesum(vec):
        for p in perms:
            vec = vec + lax.gather(vec, p, _gd, (1,),
                                   mode=lax.GatherScatterMode.PROMISE_IN_BOUNDS)
        return vec

    def _zb_init(i, carry):
        for j in range(16):
            zb[pl.ds(i * 256 + j * 16, 16)] = zero16
        return carry
    lax.fori_loop(0, ZB // 256, _zb_init, 0)

    tbase = b * (H * WST) + t * SEG

    def _zissue(i, carry):
        pltpu.async_copy(zb, w_hbm.at[pl.ds(tbase + i * ZB, ZB)], semz)
        return carry
    lax.fori_loop(0, NZD, _zissue, 0)
    pltpu.async_copy(zb.at[pl.ds(0, ZREM)],
                     w_hbm.at[pl.ds(tbase + NZD * ZB, ZREM)], semz)

    rowbase = b * (E // 128) + t * NCH
    pltpu.sync_copy(src_hbm.at[pl.ds(rowbase, NCH)], srcb)
    pltpu.sync_copy(dst_hbm.at[pl.ds(rowbase, NCH)], dstb)
    pltpu.sync_copy(c_hbm.at[pl.ds(rowbase, NCH)], cb)

    def _mkkeys(r, carry):
        for j in range(8):
            sl = pl.ds(j * 16, 16)
            s16 = srcb[r, sl]
            d16 = dstb[r, sl]
            keyb[r, sl] = s16 * N + d16 + b * KST
            myidb[r, sl] = t * EPT + r * 128 + j * 16 + lane
        return carry
    lax.fori_loop(0, NCH, _mkkeys, 0)

    def _issue_scatter(idx_ref, val_ref):
        def _is(r, carry):
            pltpu.async_copy(val_ref.at[r], win_hbm.at[idx_ref.at[r]], sem1)
            return carry
        lax.fori_loop(0, NCH, _is, 0)

        def _dr(r, carry):
            pltpu.make_async_copy(val_ref.at[0], win_hbm.at[idx_ref.at[0]],
                                  sem1).wait()
            return carry
        lax.fori_loop(0, NCH, _dr, 0)

    def _issue_gather():
        def _ig(r, carry):
            pltpu.async_copy(win_hbm.at[keyb.at[r]], widb.at[r], sem1)
            return carry
        lax.fori_loop(0, NCH, _ig, 0)

        def _dr(r, carry):
            pltpu.make_async_copy(win_hbm.at[keyb.at[0]], widb.at[0],
                                  sem1).wait()
            return carry
        lax.fori_loop(0, NCH, _dr, 0)

    _issue_scatter(keyb, myidb)
    plsc.subcore_barrier()

    for _rnd in range(3):
        _issue_gather()

        def _mkscat(r, carry):
            for j in range(8):
                sl = pl.ds(j * 16, 16)
                wid = widb[r, sl]
                my = myidb[r, sl]
                trash = b * KST + NN + (my & (N - 1))
                scat[r, sl] = jnp.where(wid < my, keyb[r, sl], trash)
            return carry
        lax.fori_loop(0, NCH, _mkscat, 0)

        _issue_scatter(scat, myidb)
        plsc.subcore_barrier()

    _issue_gather()

    def _mkwkey(r, carry):
        for j in range(8):
            sl = pl.ds(j * 16, 16)
            win = widb[r, sl] == myidb[r, sl]
            wkey = srcb[r, sl] * N + dstb[r, sl]
            trash = NN + (myidb[r, sl] & (N - 1))
            wkb[r, sl] = jnp.where(win, wkey, trash)
        return carry
    lax.fori_loop(0, NCH, _mkwkey, 0)

    for h in range(H):
        nbase = (b * H + h) * N

        def _mkgidx(r, carry):
            for j in range(8):
                sl = pl.ds(j * 16, 16)
                gq[r, sl] = srcb[r, sl] + nbase
                gk[r, sl] = dstb[r, sl] + nbase
            return carry
        lax.fori_loop(0, NCH, _mkgidx, 0)

        def _chunk(ch, carry):
            dq = pltpu.async_copy(q_hbm.at[gq.at[ch]], qr, semq)
            dk = pltpu.async_copy(k_hbm.at[gk.at[ch]], kr, semk)
            dq.wait()
            dk.wait()

            def _group(g, gcarry):
                def _edge(e, dot):
                    row = g * 16 + e
                    acc = qr[row, pl.ds(0, 16)] * kr[row, pl.ds(0, 16)]
                    for i in range(1, 16):
                        sl = pl.ds(i * 16, 16)
                        acc = acc + qr[row, sl] * kr[row, sl]
                    return jnp.where(lane == e, _lanesum(acc), dot)
                dot = lax.fori_loop(0, 16, _edge, zero16)
                cv = cb[ch, pl.ds(g * 16, 16)]
                wb[h, ch, pl.ds(g * 16, 16)] = jnp.exp(dot * cv) - 1.0
                return gcarry
            lax.fori_loop(0, 8, _group, 0)
            return carry
        lax.fori_loop(0, NCH, _chunk, 0)

    def _zdrain(i, carry):
        pltpu.make_async_copy(zb, w_hbm.at[pl.ds(tbase, ZB)], semz).wait()
        return carry
    lax.fori_loop(0, NZD, _zdrain, 0)
    pltpu.make_async_copy(zb.at[pl.ds(0, ZREM)],
                          w_hbm.at[pl.ds(tbase, ZREM)], semz).wait()
    plsc.subcore_barrier()

    for h in range(H):
        hoff = (b * H + h) * WST

        def _mkwsc(r, carry):
            for j in range(8):
                sl = pl.ds(j * 16, 16)
                scat[r, sl] = wkb[r, sl] + hoff
            return carry
        lax.fori_loop(0, NCH, _mkwsc, 0)

        def _ws(r, carry):
            pltpu.async_copy(wb.at[h, r], w_hbm.at[scat.at[r]], sem1)
            return carry
        lax.fori_loop(0, NCH, _ws, 0)

        def _wd(r, carry):
            pltpu.make_async_copy(wb.at[h, 0], w_hbm.at[scat.at[0]],
                                  sem1).wait()
            return carry
        lax.fori_loop(0, NCH, _wd, 0)


def _edge_kernel(src2, dst2, c2, qf, kf):
    mesh = plsc.VectorSubcoreMesh(core_axis_name="c", subcore_axis_name="s")
    f = pl.kernel(
        _edge_body,
        out_type=(jax.ShapeDtypeStruct((B * H * WST,), jnp.float32),
                  jax.ShapeDtypeStruct((B * KST,), jnp.int32)),
        mesh=mesh,
        scratch_types=[
            pltpu.VMEM((NCH, 128), jnp.int32),
            pltpu.VMEM((NCH, 128), jnp.int32),
            pltpu.VMEM((NCH, 128), jnp.float32),
            pltpu.VMEM((NCH, 128), jnp.int32),
            pltpu.VMEM((NCH, 128), jnp.int32),
            pltpu.VMEM((NCH, 128), jnp.int32),
            pltpu.VMEM((NCH, 128), jnp.int32),
            pltpu.VMEM((NCH, 128), jnp.int32),
            pltpu.VMEM((NCH, 128), jnp.int32),
            pltpu.VMEM((NCH, 128), jnp.int32),
            pltpu.VMEM((128, D), jnp.float32),
            pltpu.VMEM((128, D), jnp.float32),
            pltpu.VMEM((H, NCH, 128), jnp.float32),
            pltpu.VMEM((ZB,), jnp.float32),
            pltpu.SemaphoreType.DMA,
            pltpu.SemaphoreType.DMA,
            pltpu.SemaphoreType.DMA,
            pltpu.SemaphoreType.DMA,
        ],
    )
    return f(src2, dst2, c2, qf, kf)


BM = 256
M2 = N // BM


def _out_body(w_ref, v_ref, vt_ref, wo_ref, bo_ref, out_ref, acc_ref):
    h = pl.program_id(1)
    m = pl.program_id(2)
    wblk = w_ref[0, 0]
    vb = v_ref[0]
    p = jnp.dot(wblk, vb, preferred_element_type=jnp.float32)
    z = jnp.sum(wblk, axis=1)
    head = (p + vt_ref[0]) / (float(N) + z)[:, None]
    contrib = jnp.dot(head, wo_ref[0], preferred_element_type=jnp.float32)

    @pl.when(h == 0)
    def _():
        acc_ref[m] = contrib

    @pl.when(h > 0)
    def _():
        acc_ref[m] = acc_ref[m] + contrib

    @pl.when(h == H - 1)
    def _():
        out_ref[0] = acc_ref[m] + bo_ref[...]


def _tc2(w4, v, vt, wo3, bo2):
    return pl.pallas_call(
        _out_body,
        grid=(B, H, M2),
        in_specs=[
            pl.BlockSpec((1, 1, BM, N), lambda b, h, m: (b, h, m, 0)),
            pl.BlockSpec((1, N, D), lambda b, h, m: (b * H + h, 0, 0)),
            pl.BlockSpec((1, 1, D), lambda b, h, m: (b * H + h, 0, 0)),
            pl.BlockSpec((1, D, OUT), lambda b, h, m: (h, 0, 0)),
            pl.BlockSpec((1, OUT), lambda b, h, m: (0, 0)),
        ],
        out_specs=pl.BlockSpec((1, BM, OUT), lambda b, h, m: (b, m, 0)),
        out_shape=jax.ShapeDtypeStruct((B, N, OUT), jnp.float32),
        scratch_shapes=[pltpu.VMEM((M2, BM, OUT), jnp.float32)],
        compiler_params=pltpu.CompilerParams(
            dimension_semantics=("arbitrary", "arbitrary", "arbitrary")),
    )(w4, v, vt, wo3, bo2)


def kernel(nodefeatures, node_indices, nodeattributes,
           Wq, bq, Wk, bk, Wv, bv, Wout, bout):
    src2 = node_indices[:, 0, :].reshape(B * (E // 128), 128)
    dst2 = node_indices[:, 1, :].reshape(B * (E // 128), 128)

    c3 = _tcc(nodeattributes)
    c2 = c3.reshape(B * (E // 128), 128)

    q, k, v, vt = _tc1(nodefeatures,
                       Wq, bq.reshape(H, 1, D),
                       Wk, bk.reshape(H, 1, D),
                       Wv, bv.reshape(H, 1, D))
    qf = q.reshape(BH * N, D)
    kf = k.reshape(BH * N, D)

    wflat, _ = _edge_kernel(src2, dst2, c2, qf, kf)
    w4 = wflat.reshape(B, H, N + 1, N)

    return _tc2(w4, v, vt, Wout.reshape(H, D, OUT), bout.reshape(1, OUT))


# speedup vs baseline: 1.0134x; 1.0134x over previous
import functools

import jax
import jax.numpy as jnp
from jax import lax
from jax.experimental import pallas as pl
from jax.experimental.pallas import tpu as pltpu
from jax.experimental.pallas import tpu_sc as plsc

B, N, E = 2, 2048, 32768
D, H, DE = 256, 4, 16
OUT = 256
BH = B * H
NN = N * N
WST = (N + 1) * N
KST = (N + 1) * N
NTILE = 16
EPT = E // NTILE
NCH = EPT // 128
ZB = 16384
SEG = H * WST // NTILE
NZD = SEG // ZB
ZREM = SEG - NZD * ZB

_INV_SQRT_N = float(N) ** -0.5


BN = 512


def _proj_body(nf_ref, wq_ref, bq_ref, wk_ref, bk_ref, wv_ref, bv_ref,
               q_ref, k_ref, v_ref, vt_ref):
    x = nf_ref[0]
    q = jnp.dot(x, wq_ref[0], preferred_element_type=jnp.float32) + bq_ref[0]
    k = jnp.dot(x, wk_ref[0], preferred_element_type=jnp.float32) + bk_ref[0]
    v = jnp.dot(x, wv_ref[0], preferred_element_type=jnp.float32) + bv_ref[0]
    q_ref[0] = q
    k_ref[0] = k
    v_ref[0] = v
    m = pl.program_id(1)
    colsum = jnp.sum(v, axis=0, keepdims=True)

    @pl.when(m == 0)
    def _():
        vt_ref[0] = colsum

    @pl.when(m > 0)
    def _():
        vt_ref[0] = vt_ref[0] + colsum


def _tc1(nf, wq, bq3, wk, bk3, wv, bv3):
    grid = (BH, N // BN)
    hspec = lambda: pl.BlockSpec((1, D, D), lambda bh, m: (bh % H, 0, 0))
    bspec = lambda: pl.BlockSpec((1, 1, D), lambda bh, m: (bh % H, 0, 0))
    return pl.pallas_call(
        _proj_body,
        grid=grid,
        in_specs=[
            pl.BlockSpec((1, BN, D), lambda bh, m: (bh // H, m, 0)),
            hspec(), bspec(), hspec(), bspec(), hspec(), bspec(),
        ],
        out_specs=[
            pl.BlockSpec((1, BN, D), lambda bh, m: (bh, m, 0)),
            pl.BlockSpec((1, BN, D), lambda bh, m: (bh, m, 0)),
            pl.BlockSpec((1, BN, D), lambda bh, m: (bh, m, 0)),
            pl.BlockSpec((1, 1, D), lambda bh, m: (bh, 0, 0)),
        ],
        out_shape=[
            jax.ShapeDtypeStruct((BH, N, D), jnp.float32),
            jax.ShapeDtypeStruct((BH, N, D), jnp.float32),
            jax.ShapeDtypeStruct((BH, N, D), jnp.float32),
            jax.ShapeDtypeStruct((BH, 1, D), jnp.float32),
        ],
        compiler_params=pltpu.CompilerParams(
            dimension_semantics=("arbitrary", "arbitrary")),
    )(nf, wq, bq3, wk, bk3, wv, bv3)


def _csum_body(na_ref, c_ref):
    c_ref[0, 0] = jnp.sum(na_ref[0], axis=1) * _INV_SQRT_N


def _tcc(na):
    return pl.pallas_call(
        _csum_body,
        grid=(B,),
        in_specs=[pl.BlockSpec((1, E, DE), lambda b: (b, 0, 0))],
        out_specs=pl.BlockSpec((1, 1, E), lambda b: (b, 0, 0)),
        out_shape=jax.ShapeDtypeStruct((B, 1, E), jnp.float32),
    )(na)


def _edge_body(src_hbm, dst_hbm, c_hbm, q_hbm, k_hbm,
               w_hbm, win_hbm,
               srcb, dstb, cb, keyb, myidb, widb, scat, gq, gk, wkb,
               qr, kr, wb, zb,
               semz, sem1, semq, semk):
    b = lax.axis_index("c")
    t = lax.axis_index("s")
    lane = lax.iota(jnp.int32, 16)
    zero16 = jnp.zeros((16,), jnp.float32)
    _gd = lax.GatherDimensionNumbers(
        offset_dims=(), collapsed_slice_dims=(0,), start_index_map=(0,))
    perms = [(lane ^ m)[:, None] for m in (8, 4, 2, 1)]
    eqm = [lane == e for e in range(16)]

    def _lanesum(vec):
        for p in perms:
            vec = vec + lax.gather(vec, p, _gd, (1,),
                                   mode=lax.GatherScatterMode.PROMISE_IN_BOUNDS)
        return vec

    _cm_init = jax.named_scope("ph_init"); _cm_init.__enter__()
    def _zb_init(i, carry):
        for j in range(16):
            zb[pl.ds(i * 256 + j * 16, 16)] = zero16
        return carry
    lax.fori_loop(0, ZB // 256, _zb_init, 0)

    tbase = b * (H * WST) + t * SEG

    def _zissue(i, carry):
        pltpu.async_copy(zb, w_hbm.at[pl.ds(tbase + i * ZB, ZB)], semz)
        return carry
    lax.fori_loop(0, NZD, _zissue, 0)
    pltpu.async_copy(zb.at[pl.ds(0, ZREM)],
                     w_hbm.at[pl.ds(tbase + NZD * ZB, ZREM)], semz)

    rowbase = b * (E // 128) + t * NCH
    pltpu.sync_copy(src_hbm.at[pl.ds(rowbase, NCH)], srcb)
    pltpu.sync_copy(dst_hbm.at[pl.ds(rowbase, NCH)], dstb)
    pltpu.sync_copy(c_hbm.at[pl.ds(rowbase, NCH)], cb)

    def _mkkeys(r, carry):
        for j in range(8):
            sl = pl.ds(j * 16, 16)
            s16 = srcb[r, sl]
            d16 = dstb[r, sl]
            keyb[r, sl] = s16 * N + d16 + b * KST
            myidb[r, sl] = t * EPT + r * 128 + j * 16 + lane
        return carry
    lax.fori_loop(0, NCH, _mkkeys, 0)

    def _issue_scatter(idx_ref, val_ref):
        def _is(r, carry):
            pltpu.async_copy(val_ref.at[r], win_hbm.at[idx_ref.at[r]], sem1)
            return carry
        lax.fori_loop(0, NCH, _is, 0)

        def _dr(r, carry):
            pltpu.make_async_copy(val_ref.at[0], win_hbm.at[idx_ref.at[0]],
                                  sem1).wait()
            return carry
        lax.fori_loop(0, NCH, _dr, 0)

    def _issue_gather():
        def _ig(r, carry):
            pltpu.async_copy(win_hbm.at[keyb.at[r]], widb.at[r], sem1)
            return carry
        lax.fori_loop(0, NCH, _ig, 0)

        def _dr(r, carry):
            pltpu.make_async_copy(win_hbm.at[keyb.at[0]], widb.at[0],
                                  sem1).wait()
            return carry
        lax.fori_loop(0, NCH, _dr, 0)

    _cm_init.__exit__(None, None, None)
    _cm_ded = jax.named_scope("ph_dedupe"); _cm_ded.__enter__()
    _issue_scatter(keyb, myidb)
    plsc.subcore_barrier()

    for _rnd in range(3):
        _issue_gather()

        def _mkscat(r, carry):
            for j in range(8):
                sl = pl.ds(j * 16, 16)
                wid = widb[r, sl]
                my = myidb[r, sl]
                trash = b * KST + NN + (my & (N - 1))
                scat[r, sl] = jnp.where(wid < my, keyb[r, sl], trash)
            return carry
        lax.fori_loop(0, NCH, _mkscat, 0)

        _issue_scatter(scat, myidb)
        plsc.subcore_barrier()

    _issue_gather()

    def _mkwkey(r, carry):
        for j in range(8):
            sl = pl.ds(j * 16, 16)
            win = widb[r, sl] == myidb[r, sl]
            wkey = srcb[r, sl] * N + dstb[r, sl]
            trash = NN + (myidb[r, sl] & (N - 1))
            wkb[r, sl] = jnp.where(win, wkey, trash)
        return carry
    lax.fori_loop(0, NCH, _mkwkey, 0)

    _cm_ded.__exit__(None, None, None)
    _cm_dot = jax.named_scope("ph_dots"); _cm_dot.__enter__()
    for h in range(H):
        nbase = (b * H + h) * N

        def _mkgidx(r, carry):
            for j in range(8):
                sl = pl.ds(j * 16, 16)
                gq[r, sl] = srcb[r, sl] + nbase
                gk[r, sl] = dstb[r, sl] + nbase
            return carry
        lax.fori_loop(0, NCH, _mkgidx, 0)

        def _chunk(ch, carry):
            dq = pltpu.async_copy(q_hbm.at[gq.at[ch]], qr, semq)
            dk = pltpu.async_copy(k_hbm.at[gk.at[ch]], kr, semk)
            dq.wait()
            dk.wait()

            def _group(g, gcarry):
                def _edge(e, dot):
                    row = g * 16 + e
                    acc = qr[row, pl.ds(0, 16)] * kr[row, pl.ds(0, 16)]
                    for i in range(1, 16):
                        sl = pl.ds(i * 16, 16)
                        acc = acc + qr[row, sl] * kr[row, sl]
                    return jnp.where(lane == e, _lanesum(acc), dot)
                dot = lax.fori_loop(0, 16, _edge, zero16)
                cv = cb[ch, pl.ds(g * 16, 16)]
                wb[h, ch, pl.ds(g * 16, 16)] = jnp.exp(dot * cv) - 1.0
                return gcarry
            lax.fori_loop(0, 8, _group, 0)
            return carry
        lax.fori_loop(0, NCH, _chunk, 0)

    _cm_dot.__exit__(None, None, None)
    _cm_z = jax.named_scope("ph_zdrain"); _cm_z.__enter__()
    def _zdrain(i, carry):
        pltpu.make_async_copy(zb, w_hbm.at[pl.ds(tbase, ZB)], semz).wait()
        return carry
    lax.fori_loop(0, NZD, _zdrain, 0)
    pltpu.make_async_copy(zb.at[pl.ds(0, ZREM)],
                          w_hbm.at[pl.ds(tbase, ZREM)], semz).wait()
    plsc.subcore_barrier()

    _cm_z.__exit__(None, None, None)
    _cm_w = jax.named_scope("ph_wscat"); _cm_w.__enter__()
    for h in range(H):
        hoff = (b * H + h) * WST

        def _mkwsc(r, carry):
            for j in range(8):
                sl = pl.ds(j * 16, 16)
                scat[r, sl] = wkb[r, sl] + hoff
            return carry
        lax.fori_loop(0, NCH, _mkwsc, 0)

        def _ws(r, carry):
            pltpu.async_copy(wb.at[h, r], w_hbm.at[scat.at[r]], sem1)
            return carry
        lax.fori_loop(0, NCH, _ws, 0)

        def _wd(r, carry):
            pltpu.make_async_copy(wb.at[h, 0], w_hbm.at[scat.at[0]],
                                  sem1).wait()
            return carry
        lax.fori_loop(0, NCH, _wd, 0)
    _cm_w.__exit__(None, None, None)


def _edge_kernel(src2, dst2, c2, qf, kf):
    mesh = plsc.VectorSubcoreMesh(core_axis_name="c", subcore_axis_name="s")
    f = pl.kernel(
        _edge_body,
        out_type=(jax.ShapeDtypeStruct((B * H * WST,), jnp.float32),
                  jax.ShapeDtypeStruct((B * KST,), jnp.int32)),
        mesh=mesh,
        scratch_types=[
            pltpu.VMEM((NCH, 128), jnp.int32),
            pltpu.VMEM((NCH, 128), jnp.int32),
            pltpu.VMEM((NCH, 128), jnp.float32),
            pltpu.VMEM((NCH, 128), jnp.int32),
            pltpu.VMEM((NCH, 128), jnp.int32),
            pltpu.VMEM((NCH, 128), jnp.int32),
            pltpu.VMEM((NCH, 128), jnp.int32),
            pltpu.VMEM((NCH, 128), jnp.int32),
            pltpu.VMEM((NCH, 128), jnp.int32),
            pltpu.VMEM((NCH, 128), jnp.int32),
            pltpu.VMEM((128, D), jnp.float32),
            pltpu.VMEM((128, D), jnp.float32),
            pltpu.VMEM((H, NCH, 128), jnp.float32),
            pltpu.VMEM((ZB,), jnp.float32),
            pltpu.SemaphoreType.DMA,
            pltpu.SemaphoreType.DMA,
            pltpu.SemaphoreType.DMA,
            pltpu.SemaphoreType.DMA,
        ],
    )
    return f(src2, dst2, c2, qf, kf)


BM = 256
M2 = N // BM


def _out_body(w_ref, v_ref, vt_ref, wo_ref, bo_ref, out_ref, acc_ref):
    h = pl.program_id(1)
    m = pl.program_id(2)
    wblk = w_ref[0, 0]
    vb = v_ref[0]
    p = jnp.dot(wblk, vb, preferred_element_type=jnp.float32)
    z = jnp.sum(wblk, axis=1)
    head = (p + vt_ref[0]) / (float(N) + z)[:, None]
    contrib = jnp.dot(head, wo_ref[0], preferred_element_type=jnp.float32)

    @pl.when(h == 0)
    def _():
        acc_ref[m] = contrib

    @pl.when(h > 0)
    def _():
        acc_ref[m] = acc_ref[m] + contrib

    @pl.when(h == H - 1)
    def _():
        out_ref[0] = acc_ref[m] + bo_ref[...]


def _tc2(w4, v, vt, wo3, bo2):
    return pl.pallas_call(
        _out_body,
        grid=(B, H, M2),
        in_specs=[
            pl.BlockSpec((1, 1, BM, N), lambda b, h, m: (b, h, m, 0)),
            pl.BlockSpec((1, N, D), lambda b, h, m: (b * H + h, 0, 0)),
            pl.BlockSpec((1, 1, D), lambda b, h, m: (b * H + h, 0, 0)),
            pl.BlockSpec((1, D, OUT), lambda b, h, m: (h, 0, 0)),
            pl.BlockSpec((1, OUT), lambda b, h, m: (0, 0)),
        ],
        out_specs=pl.BlockSpec((1, BM, OUT), lambda b, h, m: (b, m, 0)),
        out_shape=jax.ShapeDtypeStruct((B, N, OUT), jnp.float32),
        scratch_shapes=[pltpu.VMEM((M2, BM, OUT), jnp.float32)],
        compiler_params=pltpu.CompilerParams(
            dimension_semantics=("arbitrary", "arbitrary", "arbitrary")),
    )(w4, v, vt, wo3, bo2)


def kernel(nodefeatures, node_indices, nodeattributes,
           Wq, bq, Wk, bk, Wv, bv, Wout, bout):
    src2 = node_indices[:, 0, :].reshape(B * (E // 128), 128)
    dst2 = node_indices[:, 1, :].reshape(B * (E // 128), 128)

    c3 = _tcc(nodeattributes)
    c2 = c3.reshape(B * (E // 128), 128)

    q, k, v, vt = _tc1(nodefeatures,
                       Wq, bq.reshape(H, 1, D),
                       Wk, bk.reshape(H, 1, D),
                       Wv, bv.reshape(H, 1, D))
    qf = q.reshape(BH * N, D)
    kf = k.reshape(BH * N, D)

    wflat, _ = _edge_kernel(src2, dst2, c2, qf, kf)
    w4 = wflat.reshape(B, H, N + 1, N)

    return _tc2(w4, v, vt, Wout.reshape(H, D, OUT), bout.reshape(1, OUT))


# speedup vs baseline: 1.4259x; 1.4071x over previous
import functools

import jax
import jax.numpy as jnp
from jax import lax
from jax.experimental import pallas as pl
from jax.experimental.pallas import tpu as pltpu
from jax.experimental.pallas import tpu_sc as plsc

B, N, E = 2, 2048, 32768
D, H, DE = 256, 4, 16
OUT = 256
BH = B * H
NN = N * N
WST = (N + 1) * N
KST = (N + 1) * N
NTILE = 16
EPT = E // NTILE
CC = 64
NCH = EPT // CC
ZB = 8192
SEG = H * WST // NTILE
NZD = SEG // ZB
ZREM = SEG - NZD * ZB

_INV_SQRT_N = float(N) ** -0.5


BN = 512


def _proj_body(nf_ref, wq_ref, bq_ref, wk_ref, bk_ref, wv_ref, bv_ref,
               q_ref, k_ref, v_ref, vt_ref):
    x = nf_ref[0]
    q = jnp.dot(x, wq_ref[0], preferred_element_type=jnp.float32) + bq_ref[0]
    k = jnp.dot(x, wk_ref[0], preferred_element_type=jnp.float32) + bk_ref[0]
    v = jnp.dot(x, wv_ref[0], preferred_element_type=jnp.float32) + bv_ref[0]
    q_ref[0] = q
    k_ref[0] = k
    v_ref[0] = v
    m = pl.program_id(1)
    colsum = jnp.sum(v, axis=0, keepdims=True)

    @pl.when(m == 0)
    def _():
        vt_ref[0] = colsum

    @pl.when(m > 0)
    def _():
        vt_ref[0] = vt_ref[0] + colsum


def _tc1(nf, wq, bq3, wk, bk3, wv, bv3):
    grid = (BH, N // BN)
    hspec = lambda: pl.BlockSpec((1, D, D), lambda bh, m: (bh % H, 0, 0))
    bspec = lambda: pl.BlockSpec((1, 1, D), lambda bh, m: (bh % H, 0, 0))
    return pl.pallas_call(
        _proj_body,
        grid=grid,
        in_specs=[
            pl.BlockSpec((1, BN, D), lambda bh, m: (bh // H, m, 0)),
            hspec(), bspec(), hspec(), bspec(), hspec(), bspec(),
        ],
        out_specs=[
            pl.BlockSpec((1, BN, D), lambda bh, m: (bh, m, 0)),
            pl.BlockSpec((1, BN, D), lambda bh, m: (bh, m, 0)),
            pl.BlockSpec((1, BN, D), lambda bh, m: (bh, m, 0)),
            pl.BlockSpec((1, 1, D), lambda bh, m: (bh, 0, 0)),
        ],
        out_shape=[
            jax.ShapeDtypeStruct((BH, N, D), jnp.float32),
            jax.ShapeDtypeStruct((BH, N, D), jnp.float32),
            jax.ShapeDtypeStruct((BH, N, D), jnp.float32),
            jax.ShapeDtypeStruct((BH, 1, D), jnp.float32),
        ],
        compiler_params=pltpu.CompilerParams(
            dimension_semantics=("arbitrary", "arbitrary")),
    )(nf, wq, bq3, wk, bk3, wv, bv3)


def _csum_body(na_ref, c_ref):
    c_ref[0, 0] = jnp.sum(na_ref[0], axis=1) * _INV_SQRT_N


def _tcc(na):
    return pl.pallas_call(
        _csum_body,
        grid=(B,),
        in_specs=[pl.BlockSpec((1, E, DE), lambda b: (b, 0, 0))],
        out_specs=pl.BlockSpec((1, 1, E), lambda b: (b, 0, 0)),
        out_shape=jax.ShapeDtypeStruct((B, 1, E), jnp.float32),
    )(na)


def _edge_body(src_hbm, dst_hbm, c_hbm, q_hbm, k_hbm,
               w_hbm, win_hbm,
               srcb, dstb, cb, gq, gk, scat, keyb, myidb, widb, scatk, wkb,
               qra, krb0, qrb, krb1, wb, zb,
               semz, semw, sem1, saq, sak, sbq, sbk):
    b = lax.axis_index("c")
    t = lax.axis_index("s")
    lane = lax.iota(jnp.int32, 16)
    zero16 = jnp.zeros((16,), jnp.float32)
    _gd = lax.GatherDimensionNumbers(
        offset_dims=(), collapsed_slice_dims=(0,), start_index_map=(0,))
    perms = [(lane ^ m)[:, None] for m in (8, 4, 2, 1)]

    def _lanesum(vec):
        for p in perms:
            vec = vec + lax.gather(vec, p, _gd, (1,),
                                   mode=lax.GatherScatterMode.PROMISE_IN_BOUNDS)
        return vec

    _cm_init = jax.named_scope("ph_init"); _cm_init.__enter__()

    def _zb_init(i, carry):
        for j in range(16):
            zb[pl.ds(i * 256 + j * 16, 16)] = zero16
        return carry
    lax.fori_loop(0, ZB // 256, _zb_init, 0)

    tbase = b * (H * WST) + t * SEG

    def _zissue(i, carry):
        pltpu.async_copy(zb, w_hbm.at[pl.ds(tbase + i * ZB, ZB)], semz)
        return carry
    lax.fori_loop(0, NZD, _zissue, 0)
    pltpu.async_copy(zb.at[pl.ds(0, ZREM)],
                     w_hbm.at[pl.ds(tbase + NZD * ZB, ZREM)], semz)

    rowbase = b * (E // 64) + t * NCH
    pltpu.sync_copy(src_hbm.at[pl.ds(rowbase, NCH)], srcb)
    pltpu.sync_copy(dst_hbm.at[pl.ds(rowbase, NCH)], dstb)
    pltpu.sync_copy(c_hbm.at[pl.ds(rowbase, NCH)], cb)

    def _mkkeys(r, carry):
        for j in range(8):
            sl = pl.ds(j * 16, 16)
            r2 = 2 * r + (j // 4)
            sl2 = pl.ds(16 * (j % 4), 16)
            keyb[r, sl] = srcb[r2, sl2] * N + dstb[r2, sl2] + b * KST
            myidb[r, sl] = t * EPT + r * 128 + j * 16 + lane
        return carry
    lax.fori_loop(0, 16, _mkkeys, 0)

    def _wt_issue_scatter(idx_ref):
        def _f(r, carry):
            pltpu.async_copy(myidb.at[r], win_hbm.at[idx_ref.at[r]], sem1)
            return carry
        lax.fori_loop(0, 16, _f, 0)

    def _wt_issue_gather():
        def _f(r, carry):
            pltpu.async_copy(win_hbm.at[keyb.at[r]], widb.at[r], sem1)
            return carry
        lax.fori_loop(0, 16, _f, 0)

    def _wt_drain_scatter():
        def _f(r, carry):
            pltpu.make_async_copy(myidb.at[0], win_hbm.at[keyb.at[0]],
                                  sem1).wait()
            return carry
        lax.fori_loop(0, 16, _f, 0)

    def _wt_drain_gather():
        def _f(r, carry):
            pltpu.make_async_copy(win_hbm.at[keyb.at[0]], widb.at[0],
                                  sem1).wait()
            return carry
        lax.fori_loop(0, 16, _f, 0)

    _wt_issue_scatter(keyb)

    _cm_init.__exit__(None, None, None)
    _cm_dot = jax.named_scope("ph_dots"); _cm_dot.__enter__()

    def _compute(qr, kr, ch, h):
        wrow = lax.div(ch, 2)
        wcol = lax.rem(ch, 2) * 64

        def _group(g, gcarry):
            def _edge(e, dot):
                row = g * 16 + e
                a0 = qr[row, pl.ds(0, 16)] * kr[row, pl.ds(0, 16)]
                a1 = qr[row, pl.ds(16, 16)] * kr[row, pl.ds(16, 16)]
                a2 = qr[row, pl.ds(32, 16)] * kr[row, pl.ds(32, 16)]
                a3 = qr[row, pl.ds(48, 16)] * kr[row, pl.ds(48, 16)]
                for i in range(4, 16, 4):
                    a0 = a0 + qr[row, pl.ds(i * 16, 16)] * kr[row, pl.ds(i * 16, 16)]
                    a1 = a1 + qr[row, pl.ds(i * 16 + 16, 16)] * kr[row, pl.ds(i * 16 + 16, 16)]
                    a2 = a2 + qr[row, pl.ds(i * 16 + 32, 16)] * kr[row, pl.ds(i * 16 + 32, 16)]
                    a3 = a3 + qr[row, pl.ds(i * 16 + 48, 16)] * kr[row, pl.ds(i * 16 + 48, 16)]
                acc = (a0 + a1) + (a2 + a3)
                return jnp.where(lane == e, _lanesum(acc), dot)
            dot = lax.fori_loop(0, 16, _edge, zero16)
            cv = cb[ch, pl.ds(g * 16, 16)]
            wb[h, wrow, pl.ds(wcol + g * 16, 16)] = jnp.exp(dot * cv) - 1.0
            return gcarry
        lax.fori_loop(0, 4, _group, 0)

    def _dots_seg(h, lo, hi):
        def _issue(ch, qr, kr, sq, sk):
            pltpu.async_copy(q_hbm.at[gq.at[ch]], qr, sq)
            pltpu.async_copy(k_hbm.at[gk.at[ch]], kr, sk)

        def _drain(qr, kr, sq, sk):
            pltpu.make_async_copy(q_hbm.at[gq.at[0]], qr, sq).wait()
            pltpu.make_async_copy(k_hbm.at[gk.at[0]], kr, sk).wait()

        _issue(lo, qra, krb0, saq, sak)

        def _pair(i, carry):
            ch = lo + i * 2
            _issue(ch + 1, qrb, krb1, sbq, sbk)
            _drain(qra, krb0, saq, sak)
            _compute(qra, krb0, ch, h)

            @pl.when(ch + 2 < hi)
            def _():
                _issue(ch + 2, qra, krb0, saq, sak)
            _drain(qrb, krb1, sbq, sbk)
            _compute(qrb, krb1, ch + 1, h)
            return carry
        lax.fori_loop(0, (hi - lo) // 2, _pair, 0)

    HM = NCH // 2

    for h in range(H):
        nbase = (b * H + h) * N

        def _mkgidx(r, carry):
            for j in range(4):
                sl = pl.ds(j * 16, 16)
                gq[r, sl] = srcb[r, sl] + nbase
                gk[r, sl] = dstb[r, sl] + nbase
            return carry
        lax.fori_loop(0, NCH, _mkgidx, 0)

        _dots_seg(h, 0, HM)

        if h == 0:
            _wt_drain_scatter()
            plsc.subcore_barrier()
            _wt_issue_gather()
        elif h == 1:
            _wt_drain_scatter()
            plsc.subcore_barrier()
            _wt_issue_gather()
        elif h == 2:
            _wt_drain_scatter()
            plsc.subcore_barrier()
            _wt_issue_gather()

        _dots_seg(h, HM, NCH)

        if h < 2:
            _wt_drain_gather()

            def _mkfix(r, carry):
                for j in range(8):
                    sl = pl.ds(j * 16, 16)
                    my = myidb[r, sl]
                    trash = b * KST + NN + (my & (N - 1))
                    scatk[r, sl] = jnp.where(widb[r, sl] < my, keyb[r, sl],
                                             trash)
                return carry
            lax.fori_loop(0, 16, _mkfix, 0)
            _wt_issue_scatter(scatk)
        elif h == 2:
            _wt_drain_gather()

            def _mkwin(r, carry):
                for j in range(8):
                    sl = pl.ds(j * 16, 16)
                    r2 = 2 * r + (j // 4)
                    sl2 = pl.ds(16 * (j % 4), 16)
                    my = myidb[r, sl]
                    win = widb[r, sl] == my
                    wkey = srcb[r2, sl2] * N + dstb[r2, sl2]
                    wkb[r, sl] = jnp.where(win, wkey, NN + (my & (N - 1)))
                return carry
            lax.fori_loop(0, 16, _mkwin, 0)

    _cm_dot.__exit__(None, None, None)
    _cm_z = jax.named_scope("ph_zdrain"); _cm_z.__enter__()

    def _zdrain(i, carry):
        pltpu.make_async_copy(zb, w_hbm.at[pl.ds(tbase, ZB)], semz).wait()
        return carry
    lax.fori_loop(0, NZD, _zdrain, 0)
    pltpu.make_async_copy(zb.at[pl.ds(0, ZREM)],
                          w_hbm.at[pl.ds(tbase, ZREM)], semz).wait()
    plsc.subcore_barrier()

    _cm_z.__exit__(None, None, None)
    _cm_w = jax.named_scope("ph_wscat"); _cm_w.__enter__()

    for h in range(H):
        hoff = (b * H + h) * WST

        def _mkwsc(r, carry):
            for j in range(8):
                sl = pl.ds(j * 16, 16)
                scat[h, r, sl] = wkb[r, sl] + hoff
            return carry
        lax.fori_loop(0, 16, _mkwsc, 0)

        def _ws(r, carry):
            pltpu.async_copy(wb.at[h, r], w_hbm.at[scat.at[h, r]], semw)
            return carry
        lax.fori_loop(0, 16, _ws, 0)

    def _wd(r, carry):
        pltpu.make_async_copy(wb.at[0, 0], w_hbm.at[scat.at[0, 0]],
                              semw).wait()
        return carry
    lax.fori_loop(0, H * 16, _wd, 0)
    _cm_w.__exit__(None, None, None)


def _edge_kernel(src2, dst2, c2, qf, kf):
    mesh = plsc.VectorSubcoreMesh(core_axis_name="c", subcore_axis_name="s")
    f = pl.kernel(
        _edge_body,
        out_type=(jax.ShapeDtypeStruct((B * H * WST,), jnp.float32),
                  jax.ShapeDtypeStruct((B * KST,), jnp.int32)),
        mesh=mesh,
        scratch_types=[
            pltpu.VMEM((NCH, 64), jnp.int32),
            pltpu.VMEM((NCH, 64), jnp.int32),
            pltpu.VMEM((NCH, 64), jnp.float32),
            pltpu.VMEM((NCH, 64), jnp.int32),
            pltpu.VMEM((NCH, 64), jnp.int32),
            pltpu.VMEM((H, 16, 128), jnp.int32),
            pltpu.VMEM((16, 128), jnp.int32),
            pltpu.VMEM((16, 128), jnp.int32),
            pltpu.VMEM((16, 128), jnp.int32),
            pltpu.VMEM((16, 128), jnp.int32),
            pltpu.VMEM((16, 128), jnp.int32),
            pltpu.VMEM((CC, D), jnp.float32),
            pltpu.VMEM((CC, D), jnp.float32),
            pltpu.VMEM((CC, D), jnp.float32),
            pltpu.VMEM((CC, D), jnp.float32),
            pltpu.VMEM((H, 16, 128), jnp.float32),
            pltpu.VMEM((ZB,), jnp.float32),
            pltpu.SemaphoreType.DMA,
            pltpu.SemaphoreType.DMA,
            pltpu.SemaphoreType.DMA,
            pltpu.SemaphoreType.DMA,
            pltpu.SemaphoreType.DMA,
            pltpu.SemaphoreType.DMA,
            pltpu.SemaphoreType.DMA,
        ],
    )
    return f(src2, dst2, c2, qf, kf)


BM = 256
M2 = N // BM


def _out_body(w_ref, v_ref, vt_ref, wo_ref, bo_ref, out_ref, acc_ref):
    h = pl.program_id(1)
    m = pl.program_id(2)
    wblk = w_ref[0, 0]
    vb = v_ref[0]
    p = jnp.dot(wblk, vb, preferred_element_type=jnp.float32)
    z = jnp.sum(wblk, axis=1)
    head = (p + vt_ref[0]) / (float(N) + z)[:, None]
    contrib = jnp.dot(head, wo_ref[0], preferred_element_type=jnp.float32)

    @pl.when(h == 0)
    def _():
        acc_ref[m] = contrib

    @pl.when(h > 0)
    def _():
        acc_ref[m] = acc_ref[m] + contrib

    @pl.when(h == H - 1)
    def _():
        out_ref[0] = acc_ref[m] + bo_ref[...]


def _tc2(w4, v, vt, wo3, bo2):
    return pl.pallas_call(
        _out_body,
        grid=(B, H, M2),
        in_specs=[
            pl.BlockSpec((1, 1, BM, N), lambda b, h, m: (b, h, m, 0)),
            pl.BlockSpec((1, N, D), lambda b, h, m: (b * H + h, 0, 0)),
            pl.BlockSpec((1, 1, D), lambda b, h, m: (b * H + h, 0, 0)),
            pl.BlockSpec((1, D, OUT), lambda b, h, m: (h, 0, 0)),
            pl.BlockSpec((1, OUT), lambda b, h, m: (0, 0)),
        ],
        out_specs=pl.BlockSpec((1, BM, OUT), lambda b, h, m: (b, m, 0)),
        out_shape=jax.ShapeDtypeStruct((B, N, OUT), jnp.float32),
        scratch_shapes=[pltpu.VMEM((M2, BM, OUT), jnp.float32)],
        compiler_params=pltpu.CompilerParams(
            dimension_semantics=("arbitrary", "arbitrary", "arbitrary")),
    )(w4, v, vt, wo3, bo2)


def kernel(nodefeatures, node_indices, nodeattributes,
           Wq, bq, Wk, bk, Wv, bv, Wout, bout):
    src2 = node_indices[:, 0, :].reshape(B * (E // 64), 64)
    dst2 = node_indices[:, 1, :].reshape(B * (E // 64), 64)

    c3 = _tcc(nodeattributes)
    c2 = c3.reshape(B * (E // 64), 64)

    q, k, v, vt = _tc1(nodefeatures,
                       Wq, bq.reshape(H, 1, D),
                       Wk, bk.reshape(H, 1, D),
                       Wv, bv.reshape(H, 1, D))
    qf = q.reshape(BH * N, D)
    kf = k.reshape(BH * N, D)

    wflat, _ = _edge_kernel(src2, dst2, c2, qf, kf)
    w4 = wflat.reshape(B, H, N + 1, N)

    return _tc2(w4, v, vt, Wout.reshape(H, D, OUT), bout.reshape(1, OUT))


# speedup vs baseline: 1.4341x; 1.0057x over previous
import functools

import jax
import jax.numpy as jnp
from jax import lax
from jax.experimental import pallas as pl
from jax.experimental.pallas import tpu as pltpu
from jax.experimental.pallas import tpu_sc as plsc

B, N, E = 2, 2048, 32768
D, H, DE = 256, 4, 16
OUT = 256
BH = B * H
NN = N * N
WST = (N + 1) * N
KST = (N + 1) * N
NTILE = 16
EPT = E // NTILE
CC = 64
NCH = EPT // CC
ZB = 8192
SEG = H * WST // NTILE
NZD = SEG // ZB
ZREM = SEG - NZD * ZB

_INV_SQRT_N = float(N) ** -0.5


BN = 512


def _proj_body(nf_ref, wq_ref, bq_ref, wk_ref, bk_ref, wv_ref, bv_ref,
               q_ref, k_ref, v_ref, vt_ref):
    x = nf_ref[0]
    q = jnp.dot(x, wq_ref[0], preferred_element_type=jnp.float32) + bq_ref[0]
    k = jnp.dot(x, wk_ref[0], preferred_element_type=jnp.float32) + bk_ref[0]
    v = jnp.dot(x, wv_ref[0], preferred_element_type=jnp.float32) + bv_ref[0]
    q_ref[0] = q
    k_ref[0] = k
    v_ref[0] = v
    m = pl.program_id(1)
    colsum = jnp.sum(v, axis=0, keepdims=True)

    @pl.when(m == 0)
    def _():
        vt_ref[0] = colsum

    @pl.when(m > 0)
    def _():
        vt_ref[0] = vt_ref[0] + colsum


def _tc1(nf, wq, bq3, wk, bk3, wv, bv3):
    grid = (BH, N // BN)
    hspec = lambda: pl.BlockSpec((1, D, D), lambda bh, m: (bh % H, 0, 0))
    bspec = lambda: pl.BlockSpec((1, 1, D), lambda bh, m: (bh % H, 0, 0))
    return pl.pallas_call(
        _proj_body,
        grid=grid,
        in_specs=[
            pl.BlockSpec((1, BN, D), lambda bh, m: (bh // H, m, 0)),
            hspec(), bspec(), hspec(), bspec(), hspec(), bspec(),
        ],
        out_specs=[
            pl.BlockSpec((1, BN, D), lambda bh, m: (bh, m, 0)),
            pl.BlockSpec((1, BN, D), lambda bh, m: (bh, m, 0)),
            pl.BlockSpec((1, BN, D), lambda bh, m: (bh, m, 0)),
            pl.BlockSpec((1, 1, D), lambda bh, m: (bh, 0, 0)),
        ],
        out_shape=[
            jax.ShapeDtypeStruct((BH, N, D), jnp.float32),
            jax.ShapeDtypeStruct((BH, N, D), jnp.float32),
            jax.ShapeDtypeStruct((BH, N, D), jnp.float32),
            jax.ShapeDtypeStruct((BH, 1, D), jnp.float32),
        ],
        compiler_params=pltpu.CompilerParams(
            dimension_semantics=("arbitrary", "arbitrary")),
    )(nf, wq, bq3, wk, bk3, wv, bv3)


def _csum_body(na_ref, c_ref):
    c_ref[0, 0] = jnp.sum(na_ref[0], axis=1) * _INV_SQRT_N


def _tcc(na):
    return pl.pallas_call(
        _csum_body,
        grid=(B,),
        in_specs=[pl.BlockSpec((1, E, DE), lambda b: (b, 0, 0))],
        out_specs=pl.BlockSpec((1, 1, E), lambda b: (b, 0, 0)),
        out_shape=jax.ShapeDtypeStruct((B, 1, E), jnp.float32),
    )(na)


def _edge_body(src_hbm, dst_hbm, c_hbm, q_hbm, k_hbm,
               w_hbm, win_hbm,
               srcb, dstb, cb, gq, gk, scat, keyb, myidb, widb, scatk, wkb,
               qra, krb0, qrb, krb1, wb, zb,
               semz, semw, sem1, saq, sak, sbq, sbk):
    b = lax.axis_index("c")
    t = lax.axis_index("s")
    lane = lax.iota(jnp.int32, 16)
    zero16 = jnp.zeros((16,), jnp.float32)
    _gd = lax.GatherDimensionNumbers(
        offset_dims=(), collapsed_slice_dims=(0,), start_index_map=(0,))
    perms = [(lane ^ m)[:, None] for m in (8, 4, 2, 1)]

    def _lanesum(vec):
        for p in perms:
            vec = vec + lax.gather(vec, p, _gd, (1,),
                                   mode=lax.GatherScatterMode.PROMISE_IN_BOUNDS)
        return vec

    _cm_init = jax.named_scope("ph_init"); _cm_init.__enter__()

    def _zb_init(i, carry):
        for j in range(16):
            zb[pl.ds(i * 256 + j * 16, 16)] = zero16
        return carry
    lax.fori_loop(0, ZB // 256, _zb_init, 0)

    tbase = b * (H * WST) + t * SEG

    def _zissue(i, carry):
        pltpu.async_copy(zb, w_hbm.at[pl.ds(tbase + i * ZB, ZB)], semz)
        return carry
    lax.fori_loop(0, NZD, _zissue, 0)
    pltpu.async_copy(zb.at[pl.ds(0, ZREM)],
                     w_hbm.at[pl.ds(tbase + NZD * ZB, ZREM)], semz)

    rowbase = b * (E // 64) + t * NCH
    pltpu.sync_copy(src_hbm.at[pl.ds(rowbase, NCH)], srcb)
    pltpu.sync_copy(dst_hbm.at[pl.ds(rowbase, NCH)], dstb)
    pltpu.sync_copy(c_hbm.at[pl.ds(rowbase, NCH)], cb)

    def _mkkeys(r, carry):
        for j in range(8):
            sl = pl.ds(j * 16, 16)
            r2 = 2 * r + (j // 4)
            sl2 = pl.ds(16 * (j % 4), 16)
            keyb[r, sl] = srcb[r2, sl2] * N + dstb[r2, sl2] + b * KST
            myidb[r, sl] = t * EPT + r * 128 + j * 16 + lane
        return carry
    lax.fori_loop(0, 16, _mkkeys, 0)

    def _wt_issue_scatter(idx_ref):
        def _f(r, carry):
            pltpu.async_copy(myidb.at[r], win_hbm.at[idx_ref.at[r]], sem1)
            return carry
        lax.fori_loop(0, 16, _f, 0)

    def _wt_issue_gather():
        def _f(r, carry):
            pltpu.async_copy(win_hbm.at[keyb.at[r]], widb.at[r], sem1)
            return carry
        lax.fori_loop(0, 16, _f, 0)

    def _wt_drain_scatter():
        def _f(r, carry):
            pltpu.make_async_copy(myidb.at[0], win_hbm.at[keyb.at[0]],
                                  sem1).wait()
            return carry
        lax.fori_loop(0, 16, _f, 0)

    def _wt_drain_gather():
        def _f(r, carry):
            pltpu.make_async_copy(win_hbm.at[keyb.at[0]], widb.at[0],
                                  sem1).wait()
            return carry
        lax.fori_loop(0, 16, _f, 0)

    _wt_issue_scatter(keyb)

    _cm_init.__exit__(None, None, None)
    _cm_dot = jax.named_scope("ph_dots"); _cm_dot.__enter__()

    def _compute(qr, kr, ch, h):
        wrow = lax.div(ch, 2)
        wcol = lax.rem(ch, 2) * 64

        def _group(g, gcarry):
            def _edge(e, dot):
                row = g * 16 + e
                a0 = qr[row, pl.ds(0, 16)] * kr[row, pl.ds(0, 16)]
                a1 = qr[row, pl.ds(16, 16)] * kr[row, pl.ds(16, 16)]
                a2 = qr[row, pl.ds(32, 16)] * kr[row, pl.ds(32, 16)]
                a3 = qr[row, pl.ds(48, 16)] * kr[row, pl.ds(48, 16)]
                for i in range(4, 16, 4):
                    a0 = a0 + qr[row, pl.ds(i * 16, 16)] * kr[row, pl.ds(i * 16, 16)]
                    a1 = a1 + qr[row, pl.ds(i * 16 + 16, 16)] * kr[row, pl.ds(i * 16 + 16, 16)]
                    a2 = a2 + qr[row, pl.ds(i * 16 + 32, 16)] * kr[row, pl.ds(i * 16 + 32, 16)]
                    a3 = a3 + qr[row, pl.ds(i * 16 + 48, 16)] * kr[row, pl.ds(i * 16 + 48, 16)]
                acc = (a0 + a1) + (a2 + a3)
                return jnp.where(lane == e, _lanesum(acc), dot)
            dot = lax.fori_loop(0, 16, _edge, zero16)
            cv = cb[ch, pl.ds(g * 16, 16)]
            wb[h, wrow, pl.ds(wcol + g * 16, 16)] = jnp.exp(dot * cv) - 1.0
            return gcarry
        lax.fori_loop(0, 4, _group, 0)

    def _dots_seg(h, lo, hi):
        def _issue(ch, qr, kr, sq, sk):
            pltpu.async_copy(q_hbm.at[gq.at[ch]], qr, sq)
            pltpu.async_copy(k_hbm.at[gk.at[ch]], kr, sk)

        def _drain(qr, kr, sq, sk):
            pltpu.make_async_copy(q_hbm.at[gq.at[0]], qr, sq).wait()
            pltpu.make_async_copy(k_hbm.at[gk.at[0]], kr, sk).wait()

        _issue(lo, qra, krb0, saq, sak)

        def _pair(i, carry):
            ch = lo + i * 2
            _issue(ch + 1, qrb, krb1, sbq, sbk)
            _drain(qra, krb0, saq, sak)
            _compute(qra, krb0, ch, h)

            @pl.when(ch + 2 < hi)
            def _():
                _issue(ch + 2, qra, krb0, saq, sak)
            _drain(qrb, krb1, sbq, sbk)
            _compute(qrb, krb1, ch + 1, h)
            return carry
        lax.fori_loop(0, (hi - lo) // 2, _pair, 0)

    HM = NCH // 2

    for h in range(H):
        nbase = (b * H + h) * N

        def _mkgidx(r, carry):
            for j in range(4):
                sl = pl.ds(j * 16, 16)
                gq[r, sl] = srcb[r, sl] + nbase
                gk[r, sl] = dstb[r, sl] + nbase
            return carry
        lax.fori_loop(0, NCH, _mkgidx, 0)

        _dots_seg(h, 0, HM)

        if h == 0:
            _wt_drain_scatter()
            plsc.subcore_barrier()
            _wt_issue_gather()
        elif h == 1:
            _wt_drain_scatter()
            plsc.subcore_barrier()
            _wt_issue_gather()
        elif h == 2:
            _wt_drain_scatter()
            plsc.subcore_barrier()
            _wt_issue_gather()

        _dots_seg(h, HM, NCH)

        if h < 2:
            _wt_drain_gather()

            def _mkfix(r, carry):
                for j in range(8):
                    sl = pl.ds(j * 16, 16)
                    my = myidb[r, sl]
                    trash = b * KST + NN + (my & (N - 1))
                    scatk[r, sl] = jnp.where(widb[r, sl] < my, keyb[r, sl],
                                             trash)
                return carry
            lax.fori_loop(0, 16, _mkfix, 0)
            _wt_issue_scatter(scatk)
        elif h == 2:
            _wt_drain_gather()

            def _mkwin(r, carry):
                for j in range(8):
                    sl = pl.ds(j * 16, 16)
                    r2 = 2 * r + (j // 4)
                    sl2 = pl.ds(16 * (j % 4), 16)
                    my = myidb[r, sl]
                    win = widb[r, sl] == my
                    wkey = srcb[r2, sl2] * N + dstb[r2, sl2]
                    wkb[r, sl] = jnp.where(win, wkey, NN + (my & (N - 1)))
                return carry
            lax.fori_loop(0, 16, _mkwin, 0)

    _cm_dot.__exit__(None, None, None)
    _cm_z = jax.named_scope("ph_zdrain"); _cm_z.__enter__()

    def _zdrain(i, carry):
        pltpu.make_async_copy(zb, w_hbm.at[pl.ds(tbase, ZB)], semz).wait()
        return carry
    lax.fori_loop(0, NZD, _zdrain, 0)
    pltpu.make_async_copy(zb.at[pl.ds(0, ZREM)],
                          w_hbm.at[pl.ds(tbase, ZREM)], semz).wait()
    plsc.subcore_barrier()

    _cm_z.__exit__(None, None, None)
    _cm_w = jax.named_scope("ph_wscat"); _cm_w.__enter__()

    for h in range(H):
        hoff = (b * H + h) * WST

        def _mkwsc(r, carry):
            for j in range(8):
                sl = pl.ds(j * 16, 16)
                scat[h, r, sl] = wkb[r, sl] + hoff
            return carry
        lax.fori_loop(0, 16, _mkwsc, 0)

        def _ws(r, carry):
            pltpu.async_copy(wb.at[h, r], w_hbm.at[scat.at[h, r]], semw)
            return carry
        lax.fori_loop(0, 16, _ws, 0)

    def _wd(r, carry):
        pltpu.make_async_copy(wb.at[0, 0], w_hbm.at[scat.at[0, 0]],
                              semw).wait()
        return carry
    lax.fori_loop(0, H * 16, _wd, 0)
    _cm_w.__exit__(None, None, None)


def _edge_kernel(src2, dst2, c2, qf, kf):
    mesh = plsc.VectorSubcoreMesh(core_axis_name="c", subcore_axis_name="s")
    f = pl.kernel(
        _edge_body,
        out_type=(jax.ShapeDtypeStruct((B * H * WST,), jnp.float32),
                  jax.ShapeDtypeStruct((B * KST,), jnp.int32)),
        mesh=mesh,
        scratch_types=[
            pltpu.VMEM((NCH, 64), jnp.int32),
            pltpu.VMEM((NCH, 64), jnp.int32),
            pltpu.VMEM((NCH, 64), jnp.float32),
            pltpu.VMEM((NCH, 64), jnp.int32),
            pltpu.VMEM((NCH, 64), jnp.int32),
            pltpu.VMEM((H, 16, 128), jnp.int32),
            pltpu.VMEM((16, 128), jnp.int32),
            pltpu.VMEM((16, 128), jnp.int32),
            pltpu.VMEM((16, 128), jnp.int32),
            pltpu.VMEM((16, 128), jnp.int32),
            pltpu.VMEM((16, 128), jnp.int32),
            pltpu.VMEM((CC, D), jnp.float32),
            pltpu.VMEM((CC, D), jnp.float32),
            pltpu.VMEM((CC, D), jnp.float32),
            pltpu.VMEM((CC, D), jnp.float32),
            pltpu.VMEM((H, 16, 128), jnp.float32),
            pltpu.VMEM((ZB,), jnp.float32),
            pltpu.SemaphoreType.DMA,
            pltpu.SemaphoreType.DMA,
            pltpu.SemaphoreType.DMA,
            pltpu.SemaphoreType.DMA,
            pltpu.SemaphoreType.DMA,
            pltpu.SemaphoreType.DMA,
            pltpu.SemaphoreType.DMA,
        ],
    )
    return f(src2, dst2, c2, qf, kf)


BM = 512
M2 = N // BM


def _out_body(w_ref, v_ref, vt_ref, wo_ref, bo_ref, out_ref, acc_ref):
    h = pl.program_id(2)
    wblk = w_ref[0, 0]
    vb = v_ref[0]
    p = jnp.dot(wblk, vb, preferred_element_type=jnp.float32)
    z = jnp.sum(wblk, axis=1)
    head = (p + vt_ref[0]) / (float(N) + z)[:, None]
    contrib = jnp.dot(head, wo_ref[0], preferred_element_type=jnp.float32)

    @pl.when(h == 0)
    def _():
        acc_ref[...] = contrib

    @pl.when(h > 0)
    def _():
        acc_ref[...] = acc_ref[...] + contrib

    @pl.when(h == H - 1)
    def _():
        out_ref[0] = acc_ref[...] + bo_ref[...]


def _tc2(w4, v, vt, wo3, bo2):
    return pl.pallas_call(
        _out_body,
        grid=(B, M2, H),
        in_specs=[
            pl.BlockSpec((1, 1, BM, N), lambda b, m, h: (b, h, m, 0)),
            pl.BlockSpec((1, N, D), lambda b, m, h: (b * H + h, 0, 0)),
            pl.BlockSpec((1, 1, D), lambda b, m, h: (b * H + h, 0, 0)),
            pl.BlockSpec((1, D, OUT), lambda b, m, h: (h, 0, 0)),
            pl.BlockSpec((1, OUT), lambda b, m, h: (0, 0)),
        ],
        out_specs=pl.BlockSpec((1, BM, OUT), lambda b, m, h: (b, m, 0)),
        out_shape=jax.ShapeDtypeStruct((B, N, OUT), jnp.float32),
        scratch_shapes=[pltpu.VMEM((BM, OUT), jnp.float32)],
        compiler_params=pltpu.CompilerParams(
            dimension_semantics=("arbitrary", "arbitrary", "arbitrary")),
    )(w4, v, vt, wo3, bo2)


def kernel(nodefeatures, node_indices, nodeattributes,
           Wq, bq, Wk, bk, Wv, bv, Wout, bout):
    src2 = node_indices[:, 0, :].reshape(B * (E // 64), 64)
    dst2 = node_indices[:, 1, :].reshape(B * (E // 64), 64)

    c3 = _tcc(nodeattributes)
    c2 = c3.reshape(B * (E // 64), 64)

    q, k, v, vt = _tc1(nodefeatures,
                       Wq, bq.reshape(H, 1, D),
                       Wk, bk.reshape(H, 1, D),
                       Wv, bv.reshape(H, 1, D))
    qf = q.reshape(BH * N, D)
    kf = k.reshape(BH * N, D)

    wflat, _ = _edge_kernel(src2, dst2, c2, qf, kf)
    w4 = wflat.reshape(B, H, N + 1, N)

    return _tc2(w4, v, vt, Wout.reshape(H, D, OUT), bout.reshape(1, OUT))
